# baseline (speedup 1.0000x reference)
import functools

import jax
import jax.numpy as jnp
from jax import lax
from jax.experimental import pallas as pl
from jax.experimental.pallas import tpu as pltpu
from jax.experimental.pallas import tpu_sc as plsc

_B = 128
_VOCAB = 1000
_EMB = 64
_NC = 2
_NS = 16
_NW = _NC * _NS


def _argmax_tokens_kernel(x_ref, out_ref):
    xb = x_ref[...]
    iota = lax.broadcasted_iota(jnp.int32, xb.shape, 2)
    m = jnp.max(xb, axis=-1, keepdims=True)
    eq = xb == m
    out_ref[...] = jnp.min(jnp.where(eq, iota, _VOCAB), axis=-1)


def _make_sc_gather(n_rows):
    b_per_w = n_rows // _NW
    mesh = plsc.VectorSubcoreMesh(core_axis_name="c", subcore_axis_name="s")

    @functools.partial(
        pl.kernel, mesh=mesh,
        out_type=jax.ShapeDtypeStruct((n_rows, _EMB), jnp.float32),
        scratch_types=[
            pltpu.VMEM((b_per_w,), jnp.int32),
            pltpu.VMEM((b_per_w, _EMB), jnp.float32),
            pltpu.SemaphoreType.DMA,
        ],
        compiler_params=pltpu.CompilerParams(use_tc_tiling_on_sc=False),
    )
    def gather(tokens_hbm, table_hbm, out_hbm, idx_v, rows_v, sem):
        wid = lax.axis_index("s") * _NC + lax.axis_index("c")
        base = wid * b_per_w
        pltpu.sync_copy(tokens_hbm.at[pl.ds(base, b_per_w)], idx_v)
        pltpu.async_copy(table_hbm.at[idx_v], rows_v, sem).wait()
        pltpu.sync_copy(rows_v, out_hbm.at[pl.ds(base, b_per_w)])

    return gather


@jax.jit
def kernel(x, dictionary_weight):
    b, s, v = x.shape
    tokens = pl.pallas_call(
        _argmax_tokens_kernel,
        grid=(b // _B,),
        in_specs=[pl.BlockSpec((_B, s, v), lambda i: (i, 0, 0))],
        out_specs=pl.BlockSpec((_B, s), lambda i: (i, 0)),
        out_shape=jax.ShapeDtypeStruct((b, s), jnp.int32),
        compiler_params=pltpu.CompilerParams(
            dimension_semantics=("parallel",),
            vmem_limit_bytes=63 * 1024 * 1024),
    )(x)
    tokens_flat = tokens.reshape(b * s)
    out = _make_sc_gather(b * s)(tokens_flat, dictionary_weight)
    return out.reshape(b, s, _EMB)


# speedup vs baseline: 1.0095x; 1.0095x over previous
import functools

import jax
import jax.numpy as jnp
from jax import lax
from jax.experimental import pallas as pl
from jax.experimental.pallas import tpu as pltpu
from jax.experimental.pallas import tpu_sc as plsc

_B = 64
_VOCAB = 1000
_EMB = 64
_NC = 2
_NS = 16
_NW = _NC * _NS


def _argmax_tokens_kernel(x_ref, out_ref):
    xb = x_ref[...]
    iota = lax.broadcasted_iota(jnp.int32, xb.shape, 2)
    m = jnp.max(xb, axis=-1, keepdims=True)
    eq = xb == m
    out_ref[...] = jnp.min(jnp.where(eq, iota, _VOCAB), axis=-1)


def _make_sc_gather(n_rows):
    b_per_w = n_rows // _NW
    mesh = plsc.VectorSubcoreMesh(core_axis_name="c", subcore_axis_name="s")

    @functools.partial(
        pl.kernel, mesh=mesh,
        out_type=jax.ShapeDtypeStruct((n_rows, _EMB), jnp.float32),
        scratch_types=[
            pltpu.VMEM((b_per_w,), jnp.int32),
            pltpu.VMEM((b_per_w, _EMB), jnp.float32),
            pltpu.SemaphoreType.DMA,
        ],
        compiler_params=pltpu.CompilerParams(use_tc_tiling_on_sc=False),
    )
    def gather(tokens_hbm, table_hbm, out_hbm, idx_v, rows_v, sem):
        wid = lax.axis_index("s") * _NC + lax.axis_index("c")
        base = wid * b_per_w
        pltpu.sync_copy(tokens_hbm.at[pl.ds(base, b_per_w)], idx_v)
        pltpu.async_copy(table_hbm.at[idx_v], rows_v, sem).wait()
        pltpu.sync_copy(rows_v, out_hbm.at[pl.ds(base, b_per_w)])

    return gather


@jax.jit
def kernel(x, dictionary_weight):
    b, s, v = x.shape
    tokens = pl.pallas_call(
        _argmax_tokens_kernel,
        grid=(b // _B,),
        in_specs=[pl.BlockSpec((_B, s, v), lambda i: (i, 0, 0))],
        out_specs=pl.BlockSpec((_B, s), lambda i: (i, 0)),
        out_shape=jax.ShapeDtypeStruct((b, s), jnp.int32),
        compiler_params=pltpu.CompilerParams(
            dimension_semantics=("parallel",)),
    )(x)
    tokens_flat = tokens.reshape(b * s)
    out = _make_sc_gather(b * s)(tokens_flat, dictionary_weight)
    return out.reshape(b, s, _EMB)
